# baseline (speedup 1.0000x reference)
import functools
import math

import jax
import jax.numpy as jnp
from jax import lax
from jax.experimental import pallas as pl
from jax.experimental.pallas import tpu as pltpu
from jax.experimental.pallas import tpu_sc as plsc

_N = 16384
_SIZE = 1000
_SMOOTH = 0.1
_CONF = 1.0 - _SMOOTH
_S = _SMOOTH / (_SIZE - 2)
_K = (_SIZE - 2) * _S * math.log(_S) + _CONF * math.log(_CONF)

_INFO = plsc.get_sparse_core_info()
_NC, _NS, _L = _INFO.num_cores, _INFO.num_subcores, _INFO.num_lanes
_NW = _NC * _NS
_RPW = _N // _NW
_TR = 32
_NG = _TR // _L
_NT = _RPW // _TR
_UNROLL = 8


@functools.partial(
    pl.kernel,
    mesh=plsc.VectorSubcoreMesh(core_axis_name="c", subcore_axis_name="s"),
    out_type=jax.ShapeDtypeStruct((_NW * _L,), jnp.float32),
    scratch_types=[
        pltpu.VMEM((_RPW,), jnp.int32),
        pltpu.VMEM((_TR, _SIZE), jnp.float32),
        pltpu.VMEM((_TR, _SIZE), jnp.float32),
        pltpu.VMEM((_L,), jnp.float32),
        pltpu.VMEM((_L,), jnp.float32),
        pltpu.VMEM((_L,), jnp.float32),
        pltpu.VMEM((_L,), jnp.float32),
        pltpu.VMEM((_L,), jnp.float32),
        pltpu.SemaphoreType.DMA,
        pltpu.SemaphoreType.DMA,
    ],
    compiler_params=pltpu.CompilerParams(
        use_tc_tiling_on_sc=True, needs_layout_passes=False),
)
def _sc_loss(x_hbm, tgt_hbm, out_hbm, tgt_v, buf0, buf1,
             acc_rs_r, acc_pick_r, acc_x0_r, acc_cnt_r, res_v,
             sem0, sem1):
    wid = lax.axis_index("s") * _NC + lax.axis_index("c")
    base = wid * _RPW
    pltpu.sync_copy(tgt_hbm.at[pl.ds(base, _RPW)], tgt_v)
    iota16 = lax.iota(jnp.int32, _L)
    zeros16 = jnp.zeros((_L,), jnp.int32)
    zf = jnp.zeros((_L,), jnp.float32)
    acc_rs_r[...] = zf
    acc_pick_r[...] = zf
    acc_x0_r[...] = zf
    acc_cnt_r[...] = zf

    pltpu.async_copy(x_hbm.at[pl.ds(base, _TR)], buf0, sem0)

    def compute_tile(t, buf, sem):
        pltpu.make_async_copy(x_hbm.at[pl.ds(0, _TR)], buf, sem).wait()
        for g in range(_NG):
            rbase = g * _L

            def col_body(cb, rs):
                for u in range(_UNROLL):
                    c = cb * _UNROLL + u
                    cs = jnp.full((_L,), c, jnp.int32)
                    rs = rs + plsc.load_gather(buf, [rbase + iota16, cs])
                return rs

            rowsum = lax.fori_loop(0, _SIZE // _UNROLL, col_body, zf)
            tgt16 = tgt_v[pl.ds(t * _TR + rbase, _L)]
            valid = tgt16 != 0
            pick = plsc.load_gather(buf, [rbase + iota16, tgt16])
            x0v = plsc.load_gather(buf, [rbase + iota16, zeros16])
            one = jnp.full((_L,), 1.0, jnp.float32)
            acc_rs_r[...] = acc_rs_r[...] + jnp.where(valid, rowsum, zf)
            acc_pick_r[...] = acc_pick_r[...] + jnp.where(valid, pick, zf)
            acc_x0_r[...] = acc_x0_r[...] + jnp.where(valid, x0v, zf)
            acc_cnt_r[...] = acc_cnt_r[...] + jnp.where(valid, one, zf)

    def tile_body(t, carry):
        nxt = t + 1
        parity = lax.rem(t, 2)

        @pl.when(jnp.logical_and(nxt < _NT, lax.rem(nxt, 2) == 1))
        def _start1():
            pltpu.async_copy(x_hbm.at[pl.ds(base + nxt * _TR, _TR)], buf1, sem1)

        @pl.when(jnp.logical_and(nxt < _NT, lax.rem(nxt, 2) == 0))
        def _start0():
            pltpu.async_copy(x_hbm.at[pl.ds(base + nxt * _TR, _TR)], buf0, sem0)

        @pl.when(parity == 0)
        def _c0():
            compute_tile(t, buf0, sem0)

        @pl.when(parity == 1)
        def _c1():
            compute_tile(t, buf1, sem1)

        return carry

    lax.fori_loop(0, _NT, tile_body, 0)

    acc_rs = acc_rs_r[...]
    acc_pick = acc_pick_r[...]
    acc_x0 = acc_x0_r[...]
    acc_cnt = acc_cnt_r[...]
    res_v[...] = (_K * acc_cnt - _S * acc_rs + _S * acc_x0
                  - (_CONF - _S) * acc_pick)
    pltpu.sync_copy(res_v, out_hbm.at[pl.ds(wid * _L, _L)])


def kernel(x, target):
    parts = _sc_loss(x, target.astype(jnp.int32))
    return jnp.sum(parts)


# speedup vs baseline: 1.8975x; 1.8975x over previous
import functools
import math

import jax
import jax.numpy as jnp
from jax import lax
from jax.experimental import pallas as pl
from jax.experimental.pallas import tpu as pltpu
from jax.experimental.pallas import tpu_sc as plsc

_N = 16384
_SIZE = 1000
_SMOOTH = 0.1
_CONF = 1.0 - _SMOOTH
_S = _SMOOTH / (_SIZE - 2)
_K = (_SIZE - 2) * _S * math.log(_S) + _CONF * math.log(_CONF)

_INFO = plsc.get_sparse_core_info()
_NC, _NS, _L = _INFO.num_cores, _INFO.num_subcores, _INFO.num_lanes
_NW = _NC * _NS

_N_TC = 11264
_M_SC = _N - _N_TC
_TCB = 512
_TR = 32
_NG = _TR // _L
_RPW = _M_SC // _NW
_NT = _RPW // _TR
_UNROLL = 8


def _tc_body(x_ref, tgt_ref, out_ref):
    i = pl.program_id(0)

    @pl.when(i == 0)
    def _init():
        out_ref[...] = jnp.zeros((1, 1), jnp.float32)

    x = x_ref[...]
    tgt = tgt_ref[...]
    valid = (tgt != 0)
    rowsum = jnp.sum(x, axis=1, keepdims=True)
    x0 = x[:, 0:1]
    cols = jax.lax.broadcasted_iota(jnp.int32, x.shape, 1)
    pick = jnp.sum(jnp.where(cols == tgt, x, 0.0), axis=1, keepdims=True)
    per_row = _K - _S * rowsum + _S * x0 - (_CONF - _S) * pick
    out_ref[...] += jnp.sum(jnp.where(valid, per_row, 0.0)).reshape(1, 1)


@functools.partial(
    pl.kernel,
    mesh=plsc.VectorSubcoreMesh(core_axis_name="c", subcore_axis_name="s"),
    out_type=jax.ShapeDtypeStruct((_NW * _L,), jnp.float32),
    scratch_types=[
        pltpu.VMEM((_RPW,), jnp.int32),
        pltpu.VMEM((_TR, _SIZE), jnp.float32),
        pltpu.VMEM((_TR, _SIZE), jnp.float32),
        pltpu.VMEM((_L,), jnp.float32),
        pltpu.VMEM((_L,), jnp.float32),
        pltpu.VMEM((_L,), jnp.float32),
        pltpu.VMEM((_L,), jnp.float32),
        pltpu.VMEM((_L,), jnp.float32),
        pltpu.SemaphoreType.DMA,
        pltpu.SemaphoreType.DMA,
    ],
    compiler_params=pltpu.CompilerParams(
        use_tc_tiling_on_sc=True, needs_layout_passes=False),
)
def _sc_loss(x_hbm, tgt_hbm, out_hbm, tgt_v, buf0, buf1,
             acc_rs_r, acc_pick_r, acc_x0_r, acc_cnt_r, res_v,
             sem0, sem1):
    wid = lax.axis_index("s") * _NC + lax.axis_index("c")
    base = _N_TC + wid * _RPW
    pltpu.sync_copy(tgt_hbm.at[pl.ds(base, _RPW)], tgt_v)
    iota16 = lax.iota(jnp.int32, _L)
    zeros16 = jnp.zeros((_L,), jnp.int32)
    zf = jnp.zeros((_L,), jnp.float32)
    acc_rs_r[...] = zf
    acc_pick_r[...] = zf
    acc_x0_r[...] = zf
    acc_cnt_r[...] = zf

    pltpu.async_copy(x_hbm.at[pl.ds(base, _TR)], buf0, sem0)

    def compute_tile(t, buf, sem):
        pltpu.make_async_copy(x_hbm.at[pl.ds(0, _TR)], buf, sem).wait()
        for g in range(_NG):
            rbase = g * _L

            def col_body(cb, rs):
                for u in range(_UNROLL):
                    c = cb * _UNROLL + u
                    cs = jnp.full((_L,), c, jnp.int32)
                    rs = rs + plsc.load_gather(buf, [rbase + iota16, cs])
                return rs

            rowsum = lax.fori_loop(0, _SIZE // _UNROLL, col_body, zf)
            tgt16 = tgt_v[pl.ds(t * _TR + rbase, _L)]
            valid = tgt16 != 0
            pick = plsc.load_gather(buf, [rbase + iota16, tgt16])
            x0v = plsc.load_gather(buf, [rbase + iota16, zeros16])
            one = jnp.full((_L,), 1.0, jnp.float32)
            acc_rs_r[...] = acc_rs_r[...] + jnp.where(valid, rowsum, zf)
            acc_pick_r[...] = acc_pick_r[...] + jnp.where(valid, pick, zf)
            acc_x0_r[...] = acc_x0_r[...] + jnp.where(valid, x0v, zf)
            acc_cnt_r[...] = acc_cnt_r[...] + jnp.where(valid, one, zf)

    def tile_body(t, carry):
        nxt = t + 1

        @pl.when(jnp.logical_and(nxt < _NT, lax.rem(nxt, 2) == 1))
        def _start1():
            pltpu.async_copy(x_hbm.at[pl.ds(base + nxt * _TR, _TR)], buf1, sem1)

        @pl.when(jnp.logical_and(nxt < _NT, lax.rem(nxt, 2) == 0))
        def _start0():
            pltpu.async_copy(x_hbm.at[pl.ds(base + nxt * _TR, _TR)], buf0, sem0)

        parity = lax.rem(t, 2)

        @pl.when(parity == 0)
        def _c0():
            compute_tile(t, buf0, sem0)

        @pl.when(parity == 1)
        def _c1():
            compute_tile(t, buf1, sem1)

        return carry

    lax.fori_loop(0, _NT, tile_body, 0)

    res_v[...] = (_K * acc_cnt_r[...] - _S * acc_rs_r[...]
                  + _S * acc_x0_r[...] - (_CONF - _S) * acc_pick_r[...])
    pltpu.sync_copy(res_v, out_hbm.at[pl.ds(wid * _L, _L)])


def kernel(x, target):
    tgt32 = target.astype(jnp.int32)
    sc_parts = _sc_loss(x, tgt32)
    tc_out = pl.pallas_call(
        _tc_body,
        grid=(_N_TC // _TCB,),
        in_specs=[
            pl.BlockSpec((_TCB, _SIZE), lambda i: (i, 0)),
            pl.BlockSpec((_TCB, 1), lambda i: (i, 0)),
        ],
        out_specs=pl.BlockSpec((1, 1), lambda i: (0, 0)),
        out_shape=jax.ShapeDtypeStruct((1, 1), jnp.float32),
        compiler_params=pltpu.CompilerParams(
            dimension_semantics=("arbitrary",),
        ),
    )(x, tgt32.reshape(_N, 1))
    return tc_out[0, 0] + jnp.sum(sc_parts)


# speedup vs baseline: 2.6806x; 1.4127x over previous
import math

import jax
import jax.numpy as jnp
from jax.experimental import pallas as pl
from jax.experimental.pallas import tpu as pltpu

_N = 16384
_SIZE = 1000
_SMOOTH = 0.1
_CONF = 1.0 - _SMOOTH
_S = _SMOOTH / (_SIZE - 2)
_K = (_SIZE - 2) * _S * math.log(_S) + _CONF * math.log(_CONF)

_ROWS_PER_BLOCK = 512
_GRID = _N // _ROWS_PER_BLOCK


def _tc_body(x_ref, tgt_ref, out_ref):
    i = pl.program_id(0)

    @pl.when(i == 0)
    def _init():
        out_ref[...] = jnp.zeros((1, 1), jnp.float32)

    x = x_ref[...]
    tgt = tgt_ref[...]
    valid = (tgt != 0)
    rowsum = jnp.sum(x, axis=1, keepdims=True)
    x0 = x[:, 0:1]
    cols = jax.lax.broadcasted_iota(jnp.int32, x.shape, 1)
    pick = jnp.sum(jnp.where(cols == tgt, x, 0.0), axis=1, keepdims=True)
    per_row = _K - _S * rowsum + _S * x0 - (_CONF - _S) * pick
    out_ref[...] += jnp.sum(jnp.where(valid, per_row, 0.0)).reshape(1, 1)


def kernel(x, target):
    tgt = target.astype(jnp.int32).reshape(_N, 1)
    out = pl.pallas_call(
        _tc_body,
        grid=(_GRID,),
        in_specs=[
            pl.BlockSpec((_ROWS_PER_BLOCK, _SIZE), lambda i: (i, 0)),
            pl.BlockSpec((_ROWS_PER_BLOCK, 1), lambda i: (i, 0)),
        ],
        out_specs=pl.BlockSpec((1, 1), lambda i: (0, 0)),
        out_shape=jax.ShapeDtypeStruct((1, 1), jnp.float32),
        compiler_params=pltpu.CompilerParams(
            dimension_semantics=("arbitrary",),
        ),
    )(x, tgt)
    return out[0, 0]


# speedup vs baseline: 10.6557x; 3.9751x over previous
import math

import jax
import jax.numpy as jnp
from jax.experimental import pallas as pl
from jax.experimental.pallas import tpu as pltpu

_N = 16384
_SIZE = 1000
_SMOOTH = 0.1
_CONF = 1.0 - _SMOOTH
_S = _SMOOTH / (_SIZE - 2)
_K = (_SIZE - 2) * _S * math.log(_S) + _CONF * math.log(_CONF)

_SAMPLES_PER_BLOCK = 1024
_GRID = _N // _SAMPLES_PER_BLOCK


def _tc_body(y_ref, tgt_ref, out_ref):
    i = pl.program_id(0)

    @pl.when(i == 0)
    def _init():
        out_ref[...] = jnp.zeros((1, 1), jnp.float32)

    y = y_ref[...]
    tgt = tgt_ref[...]
    valid = (tgt != 0)
    colsum = jnp.sum(y, axis=0, keepdims=True)
    x0 = y[0:1, :]
    classes = jax.lax.broadcasted_iota(jnp.int32, y.shape, 0)
    pick = jnp.sum(jnp.where(classes == tgt, y, 0.0), axis=0, keepdims=True)
    per_col = _K - _S * colsum + _S * x0 - (_CONF - _S) * pick
    out_ref[...] += jnp.sum(jnp.where(valid, per_col, 0.0)).reshape(1, 1)


def kernel(x, target):
    y = x.T
    tgt = target.astype(jnp.int32).reshape(1, _N)
    out = pl.pallas_call(
        _tc_body,
        grid=(_GRID,),
        in_specs=[
            pl.BlockSpec((_SIZE, _SAMPLES_PER_BLOCK), lambda i: (0, i)),
            pl.BlockSpec((1, _SAMPLES_PER_BLOCK), lambda i: (0, i)),
        ],
        out_specs=pl.BlockSpec((1, 1), lambda i: (0, 0)),
        out_shape=jax.ShapeDtypeStruct((1, 1), jnp.float32),
        compiler_params=pltpu.CompilerParams(
            dimension_semantics=("arbitrary",),
        ),
    )(y, tgt)
    return out[0, 0]
